# Initial kernel scaffold; baseline (speedup 1.0000x reference)
import jax
import jax.numpy as jnp
from jax.experimental import pallas as pl


def kernel(seq_len, rel_pos_emb):
    raise NotImplementedError("write your pallas kernel here")


# speedup vs baseline: 8.1398x; 8.1398x over previous
import functools

import jax
import jax.numpy as jnp
from jax import lax
from jax.experimental import pallas as pl
from jax.experimental.pallas import tpu as pltpu
from jax.experimental.pallas import tpu_sc as plsc

_SEQ = 2048
_D = 32
_TAB_ROWS = 255
_NUM_WORKERS = 32
_RPW = _SEQ // _NUM_WORKERS
_WIN = 2176
_ROW_W = _SEQ * _D
_L = 16


def _sc_body(table_hbm, out_hbm, tab_ref, fwin_ref, osem):
    wid = lax.axis_index("s") * 2 + lax.axis_index("c")
    i0 = wid * _RPW
    m_lo = 1984 - i0

    pltpu.sync_copy(table_hbm, tab_ref)

    c254_a = tab_ref[pl.ds(254 * _D, _L)]
    c254_b = tab_ref[pl.ds(254 * _D + _L, _L)]
    c0_a = tab_ref[pl.ds(0, _L)]
    c0_b = tab_ref[pl.ds(_L, _L)]

    t_b0 = jnp.maximum(1921 - m_lo, 0)
    t_b1 = 2174 - m_lo
    bidx0 = t_b1

    def fill_const_pre(t, _):
        fwin_ref[pl.ds(t * _D, _L)] = c254_a
        fwin_ref[pl.ds(t * _D + _L, _L)] = c254_b
        return _

    def fill_band(t, _):
        o = (bidx0 - t) * _D
        fwin_ref[pl.ds(t * _D, _L)] = tab_ref[pl.ds(o, _L)]
        fwin_ref[pl.ds(t * _D + _L, _L)] = tab_ref[pl.ds(o + _L, _L)]
        return _

    def fill_const_post(t, _):
        fwin_ref[pl.ds(t * _D, _L)] = c0_a
        fwin_ref[pl.ds(t * _D + _L, _L)] = c0_b
        return _

    lax.fori_loop(0, t_b0, fill_const_pre, 0)
    lax.fori_loop(t_b0, t_b1, fill_band, 0)
    lax.fori_loop(t_b1, _WIN, fill_const_post, 0)

    for g in range(0, _RPW, 8):
        copies = [
            pltpu.async_copy(
                fwin_ref.at[pl.ds((63 - r) * _D, _ROW_W)],
                out_hbm.at[pl.ds((i0 + r) * _ROW_W, _ROW_W)],
                osem,
            )
            for r in range(g, g + 8)
        ]
        for cp in copies:
            cp.wait()


def kernel(seq_len, rel_pos_emb):
    del seq_len
    mesh = plsc.VectorSubcoreMesh(core_axis_name="c", subcore_axis_name="s")
    run = functools.partial(
        pl.kernel,
        mesh=mesh,
        out_type=jax.ShapeDtypeStruct((_SEQ * _SEQ * _D,), jnp.float32),
        scratch_types=[
            pltpu.VMEM((_TAB_ROWS * _D,), jnp.float32),
            pltpu.VMEM((_WIN * _D,), jnp.float32),
            pltpu.SemaphoreType.DMA,
        ],
    )(_sc_body)
    flat = run(rel_pos_emb.reshape(-1))
    return flat.reshape(_SEQ, _SEQ, _D)
